# Initial kernel scaffold; baseline (speedup 1.0000x reference)
import jax
import jax.numpy as jnp
from jax.experimental import pallas as pl


def kernel(x, packed_indices, codebook):
    raise NotImplementedError("write your pallas kernel here")


# speedup vs baseline: 128.4437x; 128.4437x over previous
import functools

import jax
import jax.numpy as jnp
from jax import lax
from jax.experimental import pallas as pl
from jax.experimental.pallas import tpu as pltpu
from jax.experimental.pallas import tpu_sc as plsc

N = 4096
K = 4096
BS = 128
NE = 8
BATCH = 4
NC, NS, L = 2, 16, 16
NW = NC * NS
ROWS_W = N // NW
WPR = (K * 3) // 8 // 4
KBPR = K // BS
CLUSTERS = WPR // 3
RG2 = 4
NPASS = ROWS_W // (L * RG2)


def _sc_body(x_hbm, pk_hbm, cb_hbm, out_hbm, x_v, pk_v, cb_v, y_v):
    cid = lax.axis_index("c")
    sid = lax.axis_index("s")
    wid = cid * NS + sid
    r0 = wid * ROWS_W

    pltpu.sync_copy(x_hbm, x_v)
    pltpu.sync_copy(pk_hbm.at[pl.ds(r0 * WPR, ROWS_W * WPR)], pk_v)
    pltpu.sync_copy(cb_hbm.at[pl.ds(r0 * KBPR * NE, ROWS_W * KBPR * NE)], cb_v)

    lanes = lax.iota(jnp.int32, L)

    def pass_body(p, _):
        rowv = [lanes + (p * (L * RG2) + rg * L) for rg in range(RG2)]
        pkbase = [rowv[rg] * WPR for rg in range(RG2)]
        cbbase = [rowv[rg] * (KBPR * NE) for rg in range(RG2)]

        def cl_body(c, accs):
            accs = list(accs)
            base = 3 * c
            kb8 = (c >> 2) << 3
            fields = []
            cbi0 = []
            for rg in range(RG2):
                i0 = pkbase[rg] + base
                w0 = plsc.load_gather(pk_v, [i0])
                w1 = plsc.load_gather(pk_v, [i0 + 1])
                w2 = plsc.load_gather(pk_v, [i0 + 2])
                f0 = w0 & 0xFFFFFF
                f1 = lax.shift_right_logical(w0, 24) | ((w1 & 0xFFFF) << 8)
                f2 = lax.shift_right_logical(w1, 16) | ((w2 & 0xFF) << 16)
                f3 = lax.shift_right_logical(w2, 8)
                fields.append((f0, f1, f2, f3))
                cbi0.append(cbbase[rg] + kb8)
            xvecs = [[x_v[pl.ds(b * K + c * 32, L)],
                      x_v[pl.ds(b * K + c * 32 + L, L)]] for b in range(BATCH)]
            for g in range(4):
                for j in range(8):
                    jj = g * 8 + j
                    xs = [xvecs[b][jj // L][jj % L] for b in range(BATCH)]
                    for rg in range(RG2):
                        code = (fields[rg][g] >> (3 * j)) & 7
                        cv = plsc.load_gather(cb_v, [cbi0[rg] + code])
                        for b in range(BATCH):
                            i = rg * BATCH + b
                            accs[i] = accs[i] + cv * xs[b]
            return tuple(accs)

        acc0 = tuple(jnp.zeros((L,), jnp.float32) for _ in range(RG2 * BATCH))
        accs = lax.fori_loop(0, CLUSTERS, cl_body, acc0)
        for rg in range(RG2):
            for b in range(BATCH):
                y_v[pl.ds(b * ROWS_W + p * (L * RG2) + rg * L, L)] = (
                    accs[rg * BATCH + b])
        return 0

    lax.fori_loop(0, NPASS, pass_body, 0)
    for b in range(BATCH):
        pltpu.sync_copy(y_v.at[pl.ds(b * ROWS_W, ROWS_W)],
                        out_hbm.at[pl.ds(b * N + r0, ROWS_W)])


_sc_kernel = functools.partial(
    pl.kernel,
    out_type=jax.ShapeDtypeStruct((BATCH * N,), jnp.float32),
    mesh=plsc.VectorSubcoreMesh(
        core_axis_name="c", subcore_axis_name="s",
        num_cores=NC, num_subcores=NS),
    compiler_params=pltpu.CompilerParams(
        needs_layout_passes=False, use_tc_tiling_on_sc=False),
    scratch_types=[
        pltpu.VMEM((BATCH * K,), jnp.float32),
        pltpu.VMEM((ROWS_W * WPR,), jnp.int32),
        pltpu.VMEM((ROWS_W * KBPR * NE,), jnp.float32),
        pltpu.VMEM((BATCH * ROWS_W,), jnp.float32),
    ],
)(_sc_body)


@jax.jit
def kernel(x, packed_indices, codebook):
    pk_bytes = packed_indices.astype(jnp.uint8).reshape(-1, 4)
    pk_words = lax.bitcast_convert_type(pk_bytes, jnp.int32)
    y_flat = _sc_kernel(x.reshape(-1), pk_words, codebook.reshape(-1))
    return y_flat.reshape(BATCH, N)


# speedup vs baseline: 318.1290x; 2.4768x over previous
import functools

import jax
import jax.numpy as jnp
from jax import lax
from jax.experimental import pallas as pl
from jax.experimental.pallas import tpu as pltpu
from jax.experimental.pallas import tpu_sc as plsc

N = 4096
K = 4096
BS = 128
NE = 8
BATCH = 4
NC, NS, L = 2, 16, 16
NW = NC * NS
ROWS_W = N // NW
BPR = (K * 3) // 8
KBPR = K // BS
CLUSTERS = K // 32
RG2 = 2
CHUNK_R = L * RG2
NPASS = ROWS_W // CHUNK_R


def _sc_body(x_hbm, pk_hbm, cb_hbm, out_hbm, x_v, pk_v, cb_v, y_v):
    cid = lax.axis_index("c")
    sid = lax.axis_index("s")
    wid = cid * NS + sid
    r0 = wid * ROWS_W

    pltpu.sync_copy(x_hbm, x_v)
    pltpu.sync_copy(cb_hbm.at[pl.ds(r0 * KBPR * NE, ROWS_W * KBPR * NE)], cb_v)

    lanes = lax.iota(jnp.int32, L)

    def pass_body(p, _):
        pltpu.sync_copy(
            pk_hbm.at[pl.ds((r0 + p * CHUNK_R) * BPR, CHUNK_R * BPR)], pk_v)
        pkbase = [lanes * BPR + rg * (L * BPR) for rg in range(RG2)]
        cbbase = [(lanes + p * CHUNK_R + rg * L) * (KBPR * NE)
                  for rg in range(RG2)]

        def cl_body(c, accs):
            accs = list(accs)
            kb8 = (c >> 2) << 3
            cbi0 = [cbbase[rg] + kb8 for rg in range(RG2)]
            xvecs = [[x_v[pl.ds(b * K + c * 32, L)],
                      x_v[pl.ds(b * K + c * 32 + L, L)]] for b in range(BATCH)]
            for gg in range(4):
                g3 = c * 12 + gg * 3
                bits = []
                for rg in range(RG2):
                    i0 = pkbase[rg] + g3
                    b0 = plsc.load_gather(pk_v, [i0])
                    b1 = plsc.load_gather(pk_v, [i0 + 1])
                    b2 = plsc.load_gather(pk_v, [i0 + 2])
                    bits.append(b0 | (b1 << 8) | (b2 << 16))
                for j in range(8):
                    jj = gg * 8 + j
                    xs = [xvecs[b][jj // L][jj % L] for b in range(BATCH)]
                    for rg in range(RG2):
                        code = (bits[rg] >> (3 * j)) & 7
                        cv = plsc.load_gather(cb_v, [cbi0[rg] + code])
                        for b in range(BATCH):
                            i = rg * BATCH + b
                            accs[i] = accs[i] + cv * xs[b]
            return tuple(accs)

        acc0 = tuple(jnp.zeros((L,), jnp.float32) for _ in range(RG2 * BATCH))
        accs = lax.fori_loop(0, CLUSTERS, cl_body, acc0)
        for rg in range(RG2):
            for b in range(BATCH):
                y_v[pl.ds(b * ROWS_W + p * CHUNK_R + rg * L, L)] = (
                    accs[rg * BATCH + b])
        return 0

    lax.fori_loop(0, NPASS, pass_body, 0)
    for b in range(BATCH):
        pltpu.sync_copy(y_v.at[pl.ds(b * ROWS_W, ROWS_W)],
                        out_hbm.at[pl.ds(b * N + r0, ROWS_W)])


_sc_kernel = functools.partial(
    pl.kernel,
    out_type=jax.ShapeDtypeStruct((BATCH * N,), jnp.float32),
    mesh=plsc.VectorSubcoreMesh(
        core_axis_name="c", subcore_axis_name="s",
        num_cores=NC, num_subcores=NS),
    compiler_params=pltpu.CompilerParams(
        needs_layout_passes=False, use_tc_tiling_on_sc=False),
    scratch_types=[
        pltpu.VMEM((BATCH * K,), jnp.float32),
        pltpu.VMEM((CHUNK_R * BPR,), jnp.int32),
        pltpu.VMEM((ROWS_W * KBPR * NE,), jnp.float32),
        pltpu.VMEM((BATCH * ROWS_W,), jnp.float32),
    ],
)(_sc_body)


@jax.jit
def kernel(x, packed_indices, codebook):
    y_flat = _sc_kernel(x.reshape(-1), packed_indices, codebook.reshape(-1))
    return y_flat.reshape(BATCH, N)
